# baseline (speedup 1.0000x reference)
import jax
import jax.numpy as jnp
from jax import lax
from jax.experimental import pallas as pl
from jax.experimental.pallas import tpu as pltpu

B = 16
N = 10000
H = 128
F = 4
IN = 2 * H + 2 * F
G3 = 3 * H


def _body(uid_ref, iid_ref, p_ref,
          ublk_ref, iblk_ref, umem_ref, imem_ref,
          nu_ref, ni_ref, uout_ref, iout_ref,
          ue_ref, ie_ref, sem_g):
    b = pl.program_id(0)

    @pl.when(b == 0)
    def _compute():
        gath = [pltpu.make_async_copy(umem_ref.at[k, uid_ref[k]], ue_ref.at[k],
                                      sem_g) for k in range(B)]
        gath += [pltpu.make_async_copy(imem_ref.at[k, iid_ref[k]], ie_ref.at[k],
                                       sem_g) for k in range(B)]
        for c in gath:
            c.start()
        for c in gath:
            c.wait()

        ue = ue_ref[...]
        ie = ie_ref[...]
        lane = lax.broadcasted_iota(jnp.int32, (B, H), 1)
        sel_bhh = jnp.where(lane == 9, 1.0, 0.0)

        def matmul(x, w):
            return lax.dot_general(x, w, (((1,), (1,)), ((), ())),
                                   preferred_element_type=jnp.float32)

        def gru(e1, e2, k):
            wa = p_ref[2 * k * G3:(2 * k + 1) * G3, :]
            wb = p_ref[(2 * k + 1) * G3:(2 * k + 2) * G3, :]
            wf = p_ref[4 * G3 + k * G3:4 * G3 + (k + 1) * G3, :]
            xf = p_ref[6 * G3 + k * B:6 * G3 + (k + 1) * B, :]
            g = matmul(e1, wa) + matmul(e2, wb) + matmul(xf, wf)
            bhh = matmul(sel_bhh, wf)
            r = jax.nn.sigmoid(g[:, :H])
            z = jax.nn.sigmoid(g[:, H:2 * H])
            n = jnp.tanh(g[:, 2 * H:] + (r - 1.0) * bhh[:, 2 * H:])
            out = (1.0 - z) * n
            nrm = jnp.sqrt(jnp.sum(out * out, axis=1, keepdims=True))
            return out / jnp.maximum(nrm, 1e-12)

        nu_ref[...] = gru(ue, ie, 0)
        ni_ref[...] = gru(ie, ue, 1)

    uout_ref[...] = ublk_ref[...]
    iout_ref[...] = iblk_ref[...]

    uout_ref[0, pl.ds(uid_ref[b], 1), :] = nu_ref[pl.ds(b, 1), :]
    iout_ref[0, pl.ds(iid_ref[b], 1), :] = ni_ref[pl.ds(b, 1), :]


def kernel(user_ids, item_ids, user_features, item_features, user_memory,
           item_memory, W_ih_u, W_hh_u, b_ih_u, b_hh_u, W_ih_i, W_hh_i,
           b_ih_i, b_hh_i):
    del W_hh_u, W_hh_i
    zw = jnp.zeros((G3, H - 10), jnp.float32)
    zx = jnp.zeros((B, H - 10), jnp.float32)
    one = jnp.ones((B, 2), jnp.float32)
    packed = jnp.concatenate([
        W_ih_u[:, :H], W_ih_u[:, H + F:H + F + H],
        W_ih_i[:, :H], W_ih_i[:, H + F:H + F + H],
        jnp.concatenate([W_ih_u[:, H:H + F], W_ih_u[:, H + F + H:],
                         b_ih_u[:, None], b_hh_u[:, None], zw], axis=1),
        jnp.concatenate([W_ih_i[:, H:H + F], W_ih_i[:, H + F + H:],
                         b_ih_i[:, None], b_hh_i[:, None], zw], axis=1),
        jnp.concatenate([user_features, item_features, one, zx], axis=1),
        jnp.concatenate([item_features, user_features, one, zx], axis=1),
    ], axis=0)
    smem = pl.BlockSpec(memory_space=pltpu.SMEM)
    anym = pl.BlockSpec(memory_space=pltpu.MemorySpace.HBM)
    blk = pl.BlockSpec((1, N, H), lambda b: (b, 0, 0))
    f32 = jnp.float32
    return pl.pallas_call(
        _body,
        grid=(B,),
        out_shape=(
            jax.ShapeDtypeStruct((B, H), f32),
            jax.ShapeDtypeStruct((B, H), f32),
            jax.ShapeDtypeStruct((B, N, H), f32),
            jax.ShapeDtypeStruct((B, N, H), f32),
        ),
        in_specs=[smem, smem,
                  pl.BlockSpec((6 * G3 + 2 * B, H), lambda b: (0, 0)),
                  blk, blk, anym, anym],
        out_specs=(
            pl.BlockSpec((B, H), lambda b: (0, 0)),
            pl.BlockSpec((B, H), lambda b: (0, 0)),
            blk,
            blk,
        ),
        scratch_shapes=[
            pltpu.VMEM((B, H), f32),
            pltpu.VMEM((B, H), f32),
            pltpu.SemaphoreType.DMA,
        ],
    )(user_ids, item_ids, packed,
      user_memory, item_memory, user_memory, item_memory)
